# Initial kernel scaffold; baseline (speedup 1.0000x reference)
import jax
import jax.numpy as jnp
from jax.experimental import pallas as pl


def kernel(x, edge_index, Wl0, Wr0, b0, Wl1, Wr1, b1, Wl2, Wr2, b2):
    raise NotImplementedError("write your pallas kernel here")


# speedup vs baseline: 5.2979x; 5.2979x over previous
import functools

import jax
import jax.numpy as jnp
from jax import lax
from jax.experimental import pallas as pl
from jax.experimental.pallas import tpu as pltpu
from jax.experimental.pallas import tpu_sc as plsc

N = 10000
D = 128
E = 320000
NC = 2
NS = 16
NW = NC * NS
K = 128
NCH = E // K
CPW = NCH // NW
XTRA = NCH % NW
ZSEG = 624
ZLAST = N - (NS - 1) * ZSEG

_mesh = plsc.VectorSubcoreMesh(core_axis_name="c", subcore_axis_name="s")


def _staged_copy(src_at, dst_at, seg_len, stage):
  nfull = seg_len // K
  for t in range(nfull):
    pltpu.sync_copy(src_at(t * K, K), stage)
    pltpu.sync_copy(stage, dst_at(t * K, K))
  rem = seg_len - nfull * K
  if rem:
    pltpu.sync_copy(src_at(nfull * K, rem), stage.at[pl.ds(0, rem)])
    pltpu.sync_copy(stage.at[pl.ds(0, rem)], dst_at(nfull * K, rem))


def _zero_init(zsrc_hbm, sh, stage, base, seg_len):
  pltpu.sync_copy(zsrc_hbm, stage)
  for t in range(seg_len // K):
    pltpu.sync_copy(stage, sh.at[pl.ds(base + t * K, K)])
  rem = seg_len % K
  if rem:
    pltpu.sync_copy(stage.at[pl.ds(0, rem)], sh.at[pl.ds(base + (seg_len // K) * K, rem)])


def _worker_chunks(w):
  start = w * CPW + jnp.minimum(w, XTRA)
  count = CPW + jnp.where(w < XTRA, 1, 0)
  return start, count


def _sc_acc_body(g_hbm, src_hbm, dst_hbm, zrow_hbm, acc_out,
                 idx_s, idx_d, rows, acc_sh, sem):
  c = lax.axis_index("c")
  s = lax.axis_index("s")
  w = c * NS + s
  base = s * ZSEG

  @pl.when(s < NS - 1)
  def _():
    _zero_init(zrow_hbm, acc_sh, rows, base, ZSEG)

  @pl.when(s == NS - 1)
  def _():
    _zero_init(zrow_hbm, acc_sh, rows, base, ZLAST)

  plsc.subcore_barrier()

  start, count = _worker_chunks(w)

  def step(j, carry):
    chunk = start + j
    pltpu.sync_copy(src_hbm.at[chunk], idx_s)
    pltpu.sync_copy(dst_hbm.at[chunk], idx_d)
    pltpu.async_copy(g_hbm.at[idx_s], rows, sem).wait()
    pltpu.sync_copy(rows, acc_sh.at[idx_d], add=True)
    return carry

  lax.fori_loop(0, count, step, 0)
  plsc.subcore_barrier()

  @pl.when(s < NS - 1)
  def _():
    _staged_copy(lambda o, l: acc_sh.at[pl.ds(base + o, l)],
                 lambda o, l: acc_out.at[pl.ds(c * N + base + o, l)],
                 ZSEG, rows)

  @pl.when(s == NS - 1)
  def _():
    _staged_copy(lambda o, l: acc_sh.at[pl.ds(base + o, l)],
                 lambda o, l: acc_out.at[pl.ds(c * N + base + o, l)],
                 ZLAST, rows)


_sc_scatter = pl.kernel(
    _sc_acc_body,
    mesh=_mesh,
    out_type=jax.ShapeDtypeStruct((NC * N, D), jnp.float32),
    scratch_types=[
        pltpu.VMEM((K,), jnp.int32),
        pltpu.VMEM((K,), jnp.int32),
        pltpu.VMEM((K, D), jnp.float32),
        pltpu.VMEM_SHARED((N, D), jnp.float32),
        pltpu.SemaphoreType.DMA,
    ],
)


def _sc_cnt_body(dst_hbm, zcnt_hbm, ones_hbm, cnt_out,
                 idx_d, ones_v, cbuf, cnt_sh):
  c = lax.axis_index("c")
  s = lax.axis_index("s")
  w = c * NS + s
  base = s * ZSEG

  @pl.when(s < NS - 1)
  def _():
    _zero_init(zcnt_hbm, cnt_sh, cbuf, base, ZSEG)

  @pl.when(s == NS - 1)
  def _():
    _zero_init(zcnt_hbm, cnt_sh, cbuf, base, ZLAST)

  pltpu.sync_copy(ones_hbm, ones_v)
  plsc.subcore_barrier()

  start, count = _worker_chunks(w)

  def step(j, carry):
    chunk = start + j
    pltpu.sync_copy(dst_hbm.at[chunk], idx_d)
    pltpu.sync_copy(ones_v, cnt_sh.at[idx_d], add=True)
    return carry

  lax.fori_loop(0, count, step, 0)
  plsc.subcore_barrier()

  @pl.when(s < NS - 1)
  def _():
    _staged_copy(lambda o, l: cnt_sh.at[pl.ds(base + o, l)],
                 lambda o, l: cnt_out.at[pl.ds(c * N + base + o, l)],
                 ZSEG, cbuf)

  @pl.when(s == NS - 1)
  def _():
    _staged_copy(lambda o, l: cnt_sh.at[pl.ds(base + o, l)],
                 lambda o, l: cnt_out.at[pl.ds(c * N + base + o, l)],
                 ZLAST, cbuf)


_sc_cnt = pl.kernel(
    _sc_cnt_body,
    mesh=_mesh,
    out_type=jax.ShapeDtypeStruct((NC * N, D), jnp.float32),
    scratch_types=[
        pltpu.VMEM((K,), jnp.int32),
        pltpu.VMEM((K, D), jnp.float32),
        pltpu.VMEM((K, D), jnp.float32),
        pltpu.VMEM_SHARED((N, D), jnp.float32),
    ],
)


def _tc_pre_body(x_ref, wl_ref, wr_ref, b_ref, g_ref, r_ref):
  h = x_ref[...]
  g_ref[...] = jnp.dot(h, wl_ref[...], preferred_element_type=jnp.float32)
  r_ref[...] = jnp.dot(h, wr_ref[...], preferred_element_type=jnp.float32) + b_ref[...]


_tc_pre = pl.pallas_call(
    _tc_pre_body,
    out_shape=[jax.ShapeDtypeStruct((N, D), jnp.float32),
               jax.ShapeDtypeStruct((N, D), jnp.float32)],
)


def _tc_mid_body(acc_ref, cnt_ref, rp_ref, wl_ref, wr_ref, b_ref, g_ref, r_ref):
  acc = acc_ref[0] + acc_ref[1]
  cnt = cnt_ref[0] + cnt_ref[1]
  inv = 1.0 / jnp.maximum(cnt, 1.0)
  h = jnp.maximum(acc * inv + rp_ref[...], 0.0)
  g_ref[...] = jnp.dot(h, wl_ref[...], preferred_element_type=jnp.float32)
  r_ref[...] = jnp.dot(h, wr_ref[...], preferred_element_type=jnp.float32) + b_ref[...]


_tc_mid = pl.pallas_call(
    _tc_mid_body,
    out_shape=[jax.ShapeDtypeStruct((N, D), jnp.float32),
               jax.ShapeDtypeStruct((N, D), jnp.float32)],
)


def _tc_post_body(acc_ref, cnt_ref, rp_ref, out_ref):
  acc = acc_ref[0] + acc_ref[1]
  cnt = cnt_ref[0] + cnt_ref[1]
  inv = 1.0 / jnp.maximum(cnt, 1.0)
  out_ref[...] = acc * inv + rp_ref[...]


_tc_post = pl.pallas_call(
    _tc_post_body,
    out_shape=jax.ShapeDtypeStruct((N, D), jnp.float32),
)


def kernel(x, edge_index, Wl0, Wr0, b0, Wl1, Wr1, b1, Wl2, Wr2, b2):
  src_p = edge_index[0].astype(jnp.int32).reshape(NCH, K)
  dst_p = edge_index[1].astype(jnp.int32).reshape(NCH, K)
  zrow = jnp.zeros((K, D), jnp.float32)
  onesK = jnp.ones((K, D), jnp.float32)

  cnt = _sc_cnt(dst_p, zrow, onesK).reshape(NC, N, D)
  g0, r0 = _tc_pre(x, Wl0.T, Wr0.T, b0.reshape(1, D))
  acc0 = _sc_scatter(g0, src_p, dst_p, zrow).reshape(NC, N, D)
  g1, r1 = _tc_mid(acc0, cnt, r0, Wl1.T, Wr1.T, b1.reshape(1, D))
  acc1 = _sc_scatter(g1, src_p, dst_p, zrow).reshape(NC, N, D)
  g2, r2 = _tc_mid(acc1, cnt, r1, Wl2.T, Wr2.T, b2.reshape(1, D))
  acc2 = _sc_scatter(g2, src_p, dst_p, zrow).reshape(NC, N, D)
  return _tc_post(acc2, cnt, r2)
